# Initial kernel scaffold; baseline (speedup 1.0000x reference)
import jax
import jax.numpy as jnp
from jax.experimental import pallas as pl


def kernel(normed_x, x_means, W_k, W_v):
    raise NotImplementedError("write your pallas kernel here")


# speedup vs baseline: 3.2809x; 3.2809x over previous
import jax
import jax.numpy as jnp
from jax.experimental import pallas as pl
from jax.experimental.pallas import tpu as pltpu

B, L, D = 16, 576, 384
C = 1024
QK_DIM = 384
HEADS = 6
N = B * L
BLK = 1024


def _assign_kernel(x_ref, means_ref, sums_ref, counts_ref):
    i = pl.program_id(0)
    x = x_ref[...]
    nrm = jnp.sqrt(jnp.sum(x * x, axis=-1, keepdims=True))
    xn = x / jnp.maximum(nrm, 1e-12)
    dists = jax.lax.dot_general(
        xn, means_ref[...], (((1,), (1,)), ((), ())),
        preferred_element_type=jnp.float32)
    bucket = jnp.argmax(dists, axis=-1)
    onehot = (jax.lax.broadcasted_iota(jnp.int32, (BLK, C), 1)
              == bucket[:, None]).astype(jnp.float32)
    part_sums = jax.lax.dot_general(
        onehot, xn, (((0,), (0,)), ((), ())),
        preferred_element_type=jnp.float32)
    part_counts = jax.lax.dot_general(
        onehot, jnp.ones((BLK, 8), jnp.float32), (((0,), (0,)), ((), ())),
        preferred_element_type=jnp.float32)

    @pl.when(i == 0)
    def _init():
        sums_ref[...] = part_sums
        counts_ref[...] = part_counts

    @pl.when(i != 0)
    def _acc():
        sums_ref[...] += part_sums
        counts_ref[...] += part_counts


def _finalize_kernel(sums_ref, counts_ref, means_ref, wk_ref, wv_ref,
                     xg_ref, k_ref, v_ref):
    mn = means_ref[...]
    s = sums_ref[...]
    sn = s / jnp.maximum(jnp.sqrt(jnp.sum(s * s, axis=-1, keepdims=True)), 1e-12)
    empty = counts_ref[:, 0:1] == 0.0
    xg = jnp.where(empty, mn, sn)
    xg_ref[...] = xg
    k_ref[...] = jax.lax.dot_general(
        xg, wk_ref[...], (((1,), (1,)), ((), ())),
        preferred_element_type=jnp.float32)
    v_ref[...] = jax.lax.dot_general(
        xg, wv_ref[...], (((1,), (1,)), ((), ())),
        preferred_element_type=jnp.float32)


def kernel(normed_x, x_means, W_k, W_v):
    x = normed_x.reshape(N, D)
    mn = x_means / jnp.maximum(
        jnp.linalg.norm(x_means, axis=-1, keepdims=True), 1e-12)

    sums, counts = pl.pallas_call(
        _assign_kernel,
        grid=(N // BLK,),
        in_specs=[
            pl.BlockSpec((BLK, D), lambda i: (i, 0)),
            pl.BlockSpec((C, D), lambda i: (0, 0)),
        ],
        out_specs=[
            pl.BlockSpec((C, D), lambda i: (0, 0)),
            pl.BlockSpec((C, 8), lambda i: (0, 0)),
        ],
        out_shape=[
            jax.ShapeDtypeStruct((C, D), jnp.float32),
            jax.ShapeDtypeStruct((C, 8), jnp.float32),
        ],
    )(x, mn)

    xg, k, v = pl.pallas_call(
        _finalize_kernel,
        out_shape=[
            jax.ShapeDtypeStruct((C, D), jnp.float32),
            jax.ShapeDtypeStruct((C, QK_DIM), jnp.float32),
            jax.ShapeDtypeStruct((C, D), jnp.float32),
        ],
    )(sums, counts, mn, W_k, W_v)

    k = k.reshape(C, HEADS, QK_DIM // HEADS).transpose(1, 0, 2)
    v = v.reshape(C, HEADS, D // HEADS).transpose(1, 0, 2)
    return (k, v, xg)
